# baseline (speedup 1.0000x reference)
import functools

import jax
import jax.numpy as jnp
from jax import lax
from jax.experimental import pallas as pl
from jax.experimental.pallas import tpu as pltpu
from jax.experimental.pallas import tpu_sc as plsc

B = 1024
T = 200
C = 64
N = B * T

NC = 2
NS = 16
NW = NC * NS
L = 16

ROWS_PER_W = N // NW
CHUNK = 640
NCHUNK = ROWS_PER_W // CHUNK
NGROUP = CHUNK // L
IDXW = 128
NSTREAM = CHUNK // IDXW

EPS = 1e-5


def _rsqrt(v):
    i = lax.bitcast_convert_type(v, jnp.int32)
    y = lax.bitcast_convert_type(
        jnp.int32(0x5F3759DF) - lax.shift_right_arithmetic(i, 1), jnp.float32)
    half_v = v * 0.5
    for _ in range(3):
        y = y * (1.5 - half_v * y * y)
    return y


def _body(x_hbm, temb_hbm, pemb_hbm, gamma_hbm, beta_hbm, out_hbm,
          idx_v, rows_v, out_v, pemb_v, g_v, b_v, sem):
    wid = lax.axis_index("s") * NC + lax.axis_index("c")
    base = wid * ROWS_PER_W

    pltpu.sync_copy(pemb_hbm, pemb_v)
    pltpu.sync_copy(gamma_hbm, g_v)
    pltpu.sync_copy(beta_hbm, b_v)

    iota = lax.iota(jnp.int32, L)

    def chunk_body(k, carry):
        cbase = pl.multiple_of(base + k * CHUNK, CHUNK)
        pltpu.sync_copy(x_hbm.at[pl.ds(cbase, CHUNK)], idx_v)
        descs = [
            pltpu.async_copy(temb_hbm.at[idx_v.at[pl.ds(j * IDXW, IDXW)]],
                             rows_v.at[pl.ds(j * IDXW, IDXW)], sem)
            for j in range(NSTREAM)
        ]
        for d in descs:
            d.wait()

        def group_body(g, carry):
            row_idx = g * L + iota
            t_vec = lax.rem(cbase + row_idx, T)

            s = None
            s2 = None
            for c in range(C):
                cc = jnp.full((L,), c, jnp.int32)
                h = (plsc.load_gather(rows_v, [row_idx, cc])
                     + plsc.load_gather(pemb_v, [t_vec, cc]))
                s = h if s is None else s + h
                s2 = h * h if s2 is None else s2 + h * h
            mean = s * (1.0 / C)
            var = s2 * (1.0 / C) - mean * mean
            rstd = _rsqrt(var + EPS)
            mrstd = mean * rstd

            for c in range(C):
                cc = jnp.full((L,), c, jnp.int32)
                h = (plsc.load_gather(rows_v, [row_idx, cc])
                     + plsc.load_gather(pemb_v, [t_vec, cc]))
                gc = plsc.load_gather(g_v, [cc])
                bc = plsc.load_gather(b_v, [cc])
                o = (h * rstd - mrstd) * gc + bc
                plsc.store_scatter(out_v, [row_idx, cc], o)
            return carry

        lax.fori_loop(0, NGROUP, group_body, jnp.int32(0))

        pltpu.sync_copy(out_v, out_hbm.at[pl.ds(cbase, CHUNK)])
        return carry

    lax.fori_loop(0, NCHUNK, chunk_body, jnp.int32(0))


def _run(x1, temb, pemb, gamma, beta):
    mesh = plsc.VectorSubcoreMesh(
        core_axis_name="c", subcore_axis_name="s",
        num_cores=NC, num_subcores=NS)
    f = pl.kernel(
        _body,
        out_type=jax.ShapeDtypeStruct((N, C), jnp.float32),
        mesh=mesh,
        scratch_types=[
            pltpu.VMEM((CHUNK,), jnp.int32),
            pltpu.VMEM((CHUNK, C), jnp.float32),
            pltpu.VMEM((CHUNK, C), jnp.float32),
            pltpu.VMEM((T, C), jnp.float32),
            pltpu.VMEM((C,), jnp.float32),
            pltpu.VMEM((C,), jnp.float32),
            pltpu.SemaphoreType.DMA,
        ],
        compiler_params=pltpu.CompilerParams(
            needs_layout_passes=False, use_tc_tiling_on_sc=False),
    )
    return f(x1, temb, pemb, gamma, beta)


@jax.jit
def _kernel_impl(x, temb, pemb, gamma, beta):
    out = _run(x.reshape(N), temb, pemb, gamma, beta)
    return out.reshape(B, T, C)


def kernel(x, temb, pemb, gamma, beta):
    return _kernel_impl(x, temb, pemb, gamma, beta)


# speedup vs baseline: 2.4607x; 2.4607x over previous
import functools

import jax
import jax.numpy as jnp
from jax import lax
from jax.experimental import pallas as pl
from jax.experimental.pallas import tpu as pltpu
from jax.experimental.pallas import tpu_sc as plsc

B = 1024
T = 200
C = 64
N = B * T

NC = 2
NS = 16
NW = NC * NS
L = 16

ROWS_PER_W = N // NW
CHUNK = 640
NCHUNK = ROWS_PER_W // CHUNK
NGROUP = CHUNK // L
IDXW = 128
NSTREAM = CHUNK // IDXW

EPS = 1e-5


def _rsqrt(v):
    i = lax.bitcast_convert_type(v, jnp.int32)
    y = lax.bitcast_convert_type(
        jnp.int32(0x5F3759DF) - lax.shift_right_arithmetic(i, 1), jnp.float32)
    half_v = v * 0.5
    for _ in range(3):
        y = y * (1.5 - half_v * y * y)
    return y


def _body(x_hbm, temb_hbm, pemb_hbm, gamma_hbm, beta_hbm, out_hbm,
          idx_v, rows_v, out_v, pemb_v, g_v, b_v, sem):
    wid = lax.axis_index("s") * NC + lax.axis_index("c")
    base = wid * ROWS_PER_W

    pltpu.sync_copy(pemb_hbm, pemb_v)
    pltpu.sync_copy(gamma_hbm, g_v)
    pltpu.sync_copy(beta_hbm, b_v)

    iota = lax.iota(jnp.int32, L)

    def chunk_body(k, carry):
        cbase = pl.multiple_of(base + k * CHUNK, CHUNK)
        pltpu.sync_copy(x_hbm.at[pl.ds(cbase, CHUNK)], idx_v)
        descs = [
            pltpu.async_copy(temb_hbm.at[idx_v.at[pl.ds(j * IDXW, IDXW)]],
                             rows_v.at[pl.ds(j * IDXW, IDXW)], sem)
            for j in range(NSTREAM)
        ]
        for d in descs:
            d.wait()

        gv = [g_v[pl.ds(c * L, L)] for c in range(C // L)]
        bv = [b_v[pl.ds(c * L, L)] for c in range(C // L)]
        t0 = (k * CHUNK) % T

        def row_body(r, t):
            h = [rows_v[r, pl.ds(c * L, L)] + pemb_v[t, pl.ds(c * L, L)]
                 for c in range(C // L)]
            s = (h[0] + h[1]) + (h[2] + h[3])
            mean = jnp.broadcast_to(jnp.sum(s), (L,)) * (1.0 / C)
            d = [hc - mean for hc in h]
            sq = (d[0] * d[0] + d[1] * d[1]) + (d[2] * d[2] + d[3] * d[3])
            var = jnp.broadcast_to(jnp.sum(sq), (L,)) * (1.0 / C)
            rstd = _rsqrt(var + EPS)
            for c in range(C // L):
                out_v[r, pl.ds(c * L, L)] = d[c] * rstd * gv[c] + bv[c]
            return jnp.where(t == T - 1, 0, t + 1)

        lax.fori_loop(0, CHUNK, row_body, jnp.int32(t0), unroll=4)

        pltpu.sync_copy(out_v, out_hbm.at[pl.ds(cbase, CHUNK)])
        return carry

    lax.fori_loop(0, NCHUNK, chunk_body, jnp.int32(0))


def _run(x1, temb, pemb, gamma, beta):
    mesh = plsc.VectorSubcoreMesh(
        core_axis_name="c", subcore_axis_name="s",
        num_cores=NC, num_subcores=NS)
    f = pl.kernel(
        _body,
        out_type=jax.ShapeDtypeStruct((N, C), jnp.float32),
        mesh=mesh,
        scratch_types=[
            pltpu.VMEM((CHUNK,), jnp.int32),
            pltpu.VMEM((CHUNK, C), jnp.float32),
            pltpu.VMEM((CHUNK, C), jnp.float32),
            pltpu.VMEM((T, C), jnp.float32),
            pltpu.VMEM((C,), jnp.float32),
            pltpu.VMEM((C,), jnp.float32),
            pltpu.SemaphoreType.DMA,
        ],
        compiler_params=pltpu.CompilerParams(
            needs_layout_passes=False, use_tc_tiling_on_sc=False),
    )
    return f(x1, temb, pemb, gamma, beta)


@jax.jit
def _kernel_impl(x, temb, pemb, gamma, beta):
    out = _run(x.reshape(N), temb, pemb, gamma, beta)
    return out.reshape(B, T, C)


def kernel(x, temb, pemb, gamma, beta):
    return _kernel_impl(x, temb, pemb, gamma, beta)


# speedup vs baseline: 2.8807x; 1.1707x over previous
import functools

import jax
import jax.numpy as jnp
from jax import lax
from jax.experimental import pallas as pl
from jax.experimental.pallas import tpu as pltpu
from jax.experimental.pallas import tpu_sc as plsc

B = 1024
T = 200
C = 64
N = B * T

NC = 2
NS = 16
NW = NC * NS
L = 16

ROWS_PER_W = N // NW
CHUNK = 640
NCHUNK = ROWS_PER_W // CHUNK
NGROUP = CHUNK // L
IDXW = 128
NSTREAM = CHUNK // IDXW

EPS = 1e-5


def _rsqrt(v):
    i = lax.bitcast_convert_type(v, jnp.int32)
    y = lax.bitcast_convert_type(
        jnp.int32(0x5F3759DF) - lax.shift_right_arithmetic(i, 1), jnp.float32)
    half_v = v * 0.5
    for _ in range(2):
        y = y * (1.5 - half_v * y * y)
    return y


def _body(x_hbm, temb_hbm, pemb_hbm, gamma_hbm, beta_hbm, out_hbm,
          idx_v, rows_v, out_v, pemb_v, g_v, b_v, sem):
    wid = lax.axis_index("s") * NC + lax.axis_index("c")
    base = wid * ROWS_PER_W

    pltpu.sync_copy(pemb_hbm, pemb_v)
    pltpu.sync_copy(gamma_hbm, g_v)
    pltpu.sync_copy(beta_hbm, b_v)

    iota = lax.iota(jnp.int32, L)

    def chunk_body(k, carry):
        cbase = pl.multiple_of(base + k * CHUNK, CHUNK)
        pltpu.sync_copy(x_hbm.at[pl.ds(cbase, CHUNK)], idx_v)
        descs = [
            pltpu.async_copy(temb_hbm.at[idx_v.at[pl.ds(j * IDXW, IDXW)]],
                             rows_v.at[pl.ds(j * IDXW, IDXW)], sem)
            for j in range(NSTREAM)
        ]
        for d in descs:
            d.wait()

        gv = [g_v[pl.ds(c * L, L)] for c in range(C // L)]
        bv = [b_v[pl.ds(c * L, L)] for c in range(C // L)]
        t0 = (k * CHUNK) % T

        def row_body(r, t):
            h = [rows_v[r, pl.ds(c * L, L)] + pemb_v[t, pl.ds(c * L, L)]
                 for c in range(C // L)]
            s = (h[0] + h[1]) + (h[2] + h[3])
            sq = ((h[0] * h[0] + h[1] * h[1])
                  + (h[2] * h[2] + h[3] * h[3]))
            mean = jnp.broadcast_to(jnp.sum(s), (L,)) * (1.0 / C)
            ms2 = jnp.broadcast_to(jnp.sum(sq), (L,)) * (1.0 / C)
            var = ms2 - mean * mean
            rstd = _rsqrt(var + EPS)
            mr = mean * rstd
            for c in range(C // L):
                hn = h[c] * rstd - mr
                out_v[r, pl.ds(c * L, L)] = hn * gv[c] + bv[c]
            return jnp.where(t == T - 1, 0, t + 1)

        lax.fori_loop(0, CHUNK, row_body, jnp.int32(t0), unroll=4)

        pltpu.sync_copy(out_v, out_hbm.at[pl.ds(cbase, CHUNK)])
        return carry

    lax.fori_loop(0, NCHUNK, chunk_body, jnp.int32(0))


def _run(x1, temb, pemb, gamma, beta):
    mesh = plsc.VectorSubcoreMesh(
        core_axis_name="c", subcore_axis_name="s",
        num_cores=NC, num_subcores=NS)
    f = pl.kernel(
        _body,
        out_type=jax.ShapeDtypeStruct((N, C), jnp.float32),
        mesh=mesh,
        scratch_types=[
            pltpu.VMEM((CHUNK,), jnp.int32),
            pltpu.VMEM((CHUNK, C), jnp.float32),
            pltpu.VMEM((CHUNK, C), jnp.float32),
            pltpu.VMEM((T, C), jnp.float32),
            pltpu.VMEM((C,), jnp.float32),
            pltpu.VMEM((C,), jnp.float32),
            pltpu.SemaphoreType.DMA,
        ],
        compiler_params=pltpu.CompilerParams(
            needs_layout_passes=False, use_tc_tiling_on_sc=False),
    )
    return f(x1, temb, pemb, gamma, beta)


@jax.jit
def _kernel_impl(x, temb, pemb, gamma, beta):
    out = _run(x.reshape(N), temb, pemb, gamma, beta)
    return out.reshape(B, T, C)


def kernel(x, temb, pemb, gamma, beta):
    return _kernel_impl(x, temb, pemb, gamma, beta)


# speedup vs baseline: 3.1459x; 1.0921x over previous
import jax
import jax.numpy as jnp
from jax import lax
from jax.experimental import pallas as pl
from jax.experimental.pallas import tpu as pltpu
from jax.experimental.pallas import tpu_sc as plsc

B = 1024
T = 200
C = 64
N = B * T

NC = 2
NS = 16
NW = NC * NS
L = 16

ROWS_PER_W = N // NW
CHUNK = 128
NCHUNK = ROWS_PER_W // CHUNK
NB = 2
PAIRS = NCHUNK // NB

EPS = 1e-5


def _rsqrt(v):
    i = lax.bitcast_convert_type(v, jnp.int32)
    y = lax.bitcast_convert_type(
        jnp.int32(0x5F3759DF) - lax.shift_right_arithmetic(i, 1), jnp.float32)
    half_v = v * 0.5
    for _ in range(2):
        y = y * (1.5 - half_v * y * y)
    return y


def _body(x_hbm, temb_hbm, pemb_hbm, gamma_hbm, beta_hbm, out_hbm,
          idx_all, rows_v, out_v, pemb_v, g_v, b_v,
          gsem0, gsem1, wsem0, wsem1):
    gsem = (gsem0, gsem1)
    wsem = (wsem0, wsem1)
    wid = lax.axis_index("s") * NC + lax.axis_index("c")
    base = wid * ROWS_PER_W

    pltpu.sync_copy(pemb_hbm, pemb_v)
    pltpu.sync_copy(gamma_hbm, g_v)
    pltpu.sync_copy(beta_hbm, b_v)
    pltpu.sync_copy(x_hbm.at[pl.ds(pl.multiple_of(base, ROWS_PER_W),
                                   ROWS_PER_W)], idx_all)

    gv = [g_v[pl.ds(c * L, L)] for c in range(C // L)]
    bv = [b_v[pl.ds(c * L, L)] for c in range(C // L)]

    def gather_start(c, b):
        off = pl.multiple_of(c * CHUNK, CHUNK)
        return pltpu.async_copy(
            temb_hbm.at[idx_all.at[pl.ds(off, CHUNK)]], rows_v.at[b],
            gsem[b])

    def write_desc(c, b):
        off = pl.multiple_of(base + c * CHUNK, CHUNK)
        return pltpu.make_async_copy(
            out_v.at[b], out_hbm.at[pl.ds(off, CHUNK)], wsem[b])

    gather_start(0, 0)
    gather_start(1, 1)

    def pair_body(m, carry):
        for b in range(NB):
            c = m * NB + b
            off = pl.multiple_of(c * CHUNK, CHUNK)
            pltpu.make_async_copy(
                temb_hbm.at[idx_all.at[pl.ds(off, CHUNK)]], rows_v.at[b],
                gsem[b]).wait()

            @pl.when(m > 0)
            def _wait_prev():
                write_desc(c - NB, b).wait()

            t0 = lax.rem(c * CHUNK, T)

            def row_body(r, t):
                h = [rows_v[b, r, pl.ds(ci * L, L)]
                     + pemb_v[t, pl.ds(ci * L, L)]
                     for ci in range(C // L)]
                s = (h[0] + h[1]) + (h[2] + h[3])
                sq = ((h[0] * h[0] + h[1] * h[1])
                      + (h[2] * h[2] + h[3] * h[3]))
                mean = jnp.broadcast_to(jnp.sum(s), (L,)) * (1.0 / C)
                ms2 = jnp.broadcast_to(jnp.sum(sq), (L,)) * (1.0 / C)
                var = ms2 - mean * mean
                rstd = _rsqrt(var + EPS)
                mr = mean * rstd
                for ci in range(C // L):
                    hn = h[ci] * rstd - mr
                    out_v[b, r, pl.ds(ci * L, L)] = hn * gv[ci] + bv[ci]
                return jnp.where(t == T - 1, 0, t + 1)

            lax.fori_loop(0, CHUNK, row_body, t0, unroll=4)

            woff = pl.multiple_of(base + c * CHUNK, CHUNK)
            pltpu.async_copy(out_v.at[b],
                             out_hbm.at[pl.ds(woff, CHUNK)], wsem[b])

            @pl.when(c + NB < NCHUNK)
            def _prefetch():
                gather_start(c + NB, b)
        return carry

    lax.fori_loop(0, PAIRS, pair_body, jnp.int32(0))

    for b in range(NB):
        write_desc(NCHUNK - NB + b, b).wait()


def _run(x1, temb, pemb, gamma, beta):
    mesh = plsc.VectorSubcoreMesh(
        core_axis_name="c", subcore_axis_name="s",
        num_cores=NC, num_subcores=NS)
    f = pl.kernel(
        _body,
        out_type=jax.ShapeDtypeStruct((N, C), jnp.float32),
        mesh=mesh,
        scratch_types=[
            pltpu.VMEM((ROWS_PER_W,), jnp.int32),
            pltpu.VMEM((NB, CHUNK, C), jnp.float32),
            pltpu.VMEM((NB, CHUNK, C), jnp.float32),
            pltpu.VMEM((T, C), jnp.float32),
            pltpu.VMEM((C,), jnp.float32),
            pltpu.VMEM((C,), jnp.float32),
            pltpu.SemaphoreType.DMA,
            pltpu.SemaphoreType.DMA,
            pltpu.SemaphoreType.DMA,
            pltpu.SemaphoreType.DMA,
        ],
        compiler_params=pltpu.CompilerParams(
            needs_layout_passes=False, use_tc_tiling_on_sc=False),
    )
    return f(x1, temb, pemb, gamma, beta)


@jax.jit
def _kernel_impl(x, temb, pemb, gamma, beta):
    out = _run(x.reshape(N), temb, pemb, gamma, beta)
    return out.reshape(B, T, C)


def kernel(x, temb, pemb, gamma, beta):
    return _kernel_impl(x, temb, pemb, gamma, beta)
